# baseline (speedup 1.0000x reference)
import functools

import jax
import jax.numpy as jnp
from jax import lax
from jax.experimental import pallas as pl
from jax.experimental.pallas import tpu as pltpu
from jax.experimental.pallas import tpu_sc as plsc

B, S = 4096, 50
D = 300
DT = 38
BT = 32

NC, NS = 2, 16
NW = NC * NS
CB = 64

_mesh = plsc.VectorSubcoreMesh(core_axis_name="c", subcore_axis_name="s")


@functools.partial(
    pl.kernel,
    mesh=_mesh,
    out_type=jax.ShapeDtypeStruct((S, DT, BT, 8, 128), jnp.float32),
    scratch_types=[
        pltpu.VMEM((S, 128), jnp.int32),
        pltpu.VMEM((CB, D), jnp.float32),
        pltpu.VMEM((CB, D), jnp.float32),
        pltpu.VMEM((1, DT, 1, 8, 128), jnp.float32),
        pltpu.SemaphoreType.DMA,
        pltpu.SemaphoreType.DMA,
        pltpu.SemaphoreType.DMA,
    ],
    compiler_params=pltpu.CompilerParams(needs_layout_passes=False),
)
def _emb_gather(idx_hbm, table_hbm, out_hbm, idx_v, bufa, bufb, tbuf,
                gsem0, gsem1, wsem):
    w = lax.axis_index("s") * NC + lax.axis_index("c")
    pltpu.sync_copy(idx_hbm.at[pl.ds(0, S), pl.ds(w * 128, 128)], idx_v)

    bufs = (bufa, bufb)
    gsems = (gsem0, gsem1)
    bio = [jax.lax.iota(jnp.int32, 16) + 16 * g for g in range(CB // 16)]

    def fire_gathers(s, h, buf, gsem):
        def grp_body(g, carry):
            v = idx_v[s, pl.ds(h * CB + g * 16, 16)]
            for j in range(16):
                i = v[j]
                pltpu.async_copy(table_hbm.at[pl.ds(i, 1)],
                                 buf.at[pl.ds(g * 16 + j, 1)], gsem)
            return carry
        lax.fori_loop(0, CB // 16, grp_body, 0)

    def drain_gathers(buf, gsem):
        pltpu.make_async_copy(table_hbm.at[pl.ds(0, CB)], buf, gsem).wait()

    def transpose(buf, h):
        def dt_body(dt, carry):
            for di in range(8):
                d = dt * 8 + di
                dv = jnp.full((16,), d, jnp.int32)
                for g in range(CB // 16):
                    col = plsc.load_gather(buf, [bio[g], dv])
                    tbuf[0, dt, 0, di, pl.ds(h * CB + g * 16, 16)] = col
            return carry
        lax.fori_loop(0, DT - 1, dt_body, 0)
        for di in range(4):
            d = (DT - 1) * 8 + di
            dv = jnp.full((16,), d, jnp.int32)
            for g in range(CB // 16):
                col = plsc.load_gather(buf, [bio[g], dv])
                tbuf[0, DT - 1, 0, di, pl.ds(h * CB + g * 16, 16)] = col

    def out_window(s):
        return out_hbm.at[pl.ds(s, 1), pl.ds(0, DT), pl.ds(w, 1),
                          pl.ds(0, 8), pl.ds(0, 128)]

    def fire_write(s):
        pltpu.async_copy(tbuf, out_window(s), wsem)

    def drain_write():
        pltpu.make_async_copy(tbuf, out_window(0), wsem).wait()

    fire_gathers(0, 0, bufa, gsem0)
    fire_gathers(0, 1, bufb, gsem1)

    def main_body(s, carry):
        drain_gathers(bufa, gsem0)

        @pl.when(s >= 1)
        def _():
            drain_write()
        transpose(bufa, 0)

        @pl.when(s <= S - 2)
        def _():
            fire_gathers(s + 1, 0, bufa, gsem0)
        drain_gathers(bufb, gsem1)
        transpose(bufb, 1)
        fire_write(s)

        @pl.when(s <= S - 2)
        def _():
            fire_gathers(s + 1, 1, bufb, gsem1)
        return carry

    lax.fori_loop(0, S, main_body, 0)
    drain_write()


def kernel(input, table):
    idx_t = input.astype(jnp.int32).T
    z5 = _emb_gather(idx_t, table)
    t = jnp.transpose(z5, (2, 4, 0, 1, 3))
    return t.reshape(B, S, DT * 8)[:, :, :D]


# speedup vs baseline: 2.0048x; 2.0048x over previous
import functools

import jax
import jax.numpy as jnp
from jax import lax
from jax.experimental import pallas as pl
from jax.experimental.pallas import tpu as pltpu
from jax.experimental.pallas import tpu_sc as plsc

B, S = 4096, 50
D = 300
DT = 38
BT = 32

NC, NS = 2, 16
NW = NC * NS
CB = 64

_mesh = plsc.VectorSubcoreMesh(core_axis_name="c", subcore_axis_name="s")


@functools.partial(
    pl.kernel,
    mesh=_mesh,
    out_type=jax.ShapeDtypeStruct((S, DT, BT, 8, 128), jnp.float32),
    scratch_types=[
        pltpu.VMEM((S, 128), jnp.int32),
        pltpu.VMEM((CB, D), jnp.float32),
        pltpu.VMEM((CB, D), jnp.float32),
        pltpu.VMEM((1, DT, 1, 8, 128), jnp.float32),
        pltpu.SemaphoreType.DMA,
        pltpu.SemaphoreType.DMA,
        pltpu.SemaphoreType.DMA,
    ],
    compiler_params=pltpu.CompilerParams(needs_layout_passes=False),
)
def _emb_gather(idx_hbm, table_hbm, out_hbm, idx_v, bufa, bufb, tbuf,
                gsem0, gsem1, wsem):
    w = lax.axis_index("s") * NC + lax.axis_index("c")
    pltpu.sync_copy(idx_hbm.at[pl.ds(0, S), pl.ds(w * 128, 128)], idx_v)

    bufs = (bufa, bufb)
    gsems = (gsem0, gsem1)
    iota = jax.lax.iota(jnp.int32, 16)
    bio = [iota + 16 * g for g in range(CB // 16)]
    diag = [lax.bitwise_and(iota + k, 15) for k in range(16)]
    diag_hi = [lax.shift_right_logical(dg, 3) for dg in diag]
    zero16 = jnp.zeros((16,), jnp.int32)

    def fire_gathers(s, h, buf, gsem):
        def grp_body(g, carry):
            v = idx_v[s, pl.ds(h * CB + g * 16, 16)]
            for j in range(16):
                i = v[j]
                pltpu.async_copy(table_hbm.at[pl.ds(i, 1)],
                                 buf.at[pl.ds(g * 16 + j, 1)], gsem)
            return carry
        lax.fori_loop(0, CB // 16, grp_body, 0)

    def drain_gathers(buf, gsem):
        pltpu.make_async_copy(table_hbm.at[pl.ds(0, CB)], buf, gsem).wait()

    def transpose(buf, h):
        lanes = [bio[g] + h * CB for g in range(CB // 16)]

        def d0_body(dblk, carry):
            d0 = dblk * 16
            dhi = dblk * 2
            for g in range(CB // 16):
                for k in range(16):
                    dcol = d0 + diag[k]
                    col = plsc.load_gather(buf, [bio[g], dcol])
                    dtv = dhi + diag_hi[k]
                    div = lax.bitwise_and(dcol, 7)
                    plsc.store_scatter(
                        tbuf, [zero16, dtv, zero16, div, lanes[g]], col)
            return carry
        lax.fori_loop(0, (DT * 8) // 16, d0_body, 0)

    def out_window(s):
        return out_hbm.at[pl.ds(s, 1), pl.ds(0, DT), pl.ds(w, 1),
                          pl.ds(0, 8), pl.ds(0, 128)]

    def fire_write(s):
        pltpu.async_copy(tbuf, out_window(s), wsem)

    def drain_write():
        pltpu.make_async_copy(tbuf, out_window(0), wsem).wait()

    fire_gathers(0, 0, bufa, gsem0)
    fire_gathers(0, 1, bufb, gsem1)

    def main_body(s, carry):
        drain_gathers(bufa, gsem0)

        @pl.when(s >= 1)
        def _():
            drain_write()
        transpose(bufa, 0)

        @pl.when(s <= S - 2)
        def _():
            fire_gathers(s + 1, 0, bufa, gsem0)
        drain_gathers(bufb, gsem1)
        transpose(bufb, 1)
        fire_write(s)

        @pl.when(s <= S - 2)
        def _():
            fire_gathers(s + 1, 1, bufb, gsem1)
        return carry

    lax.fori_loop(0, S, main_body, 0)
    drain_write()


def kernel(input, table):
    idx_t = input.astype(jnp.int32).T
    z5 = _emb_gather(idx_t, table)
    t = jnp.transpose(z5, (2, 4, 0, 1, 3))
    return t.reshape(B, S, DT * 8)[:, :, :D]


# speedup vs baseline: 2.3679x; 1.1811x over previous
import functools

import jax
import jax.numpy as jnp
from jax import lax
from jax.experimental import pallas as pl
from jax.experimental.pallas import tpu as pltpu
from jax.experimental.pallas import tpu_sc as plsc

B, S = 4096, 50
D = 300
DT = 38
BT = 32

NC, NS = 2, 16
NW = NC * NS
CB = 64

_mesh = plsc.VectorSubcoreMesh(core_axis_name="c", subcore_axis_name="s")


@functools.partial(
    pl.kernel,
    mesh=_mesh,
    out_type=jax.ShapeDtypeStruct((S * DT * BT * 8, 128), jnp.float32),
    scratch_types=[
        pltpu.VMEM((S, 128), jnp.int32),
        pltpu.VMEM((CB, D), jnp.float32),
        pltpu.VMEM((CB, D), jnp.float32),
        pltpu.VMEM((DT * 8, 128), jnp.float32),
        pltpu.SemaphoreType.DMA,
        pltpu.SemaphoreType.DMA,
        pltpu.SemaphoreType.DMA,
    ],
    compiler_params=pltpu.CompilerParams(needs_layout_passes=False),
)
def _emb_gather(idx_hbm, table_hbm, out_hbm, idx_v, bufa, bufb, tbuf,
                gsem0, gsem1, wsem):
    w = lax.axis_index("s") * NC + lax.axis_index("c")
    pltpu.sync_copy(idx_hbm.at[pl.ds(0, S), pl.ds(w * 128, 128)], idx_v)

    bufs = (bufa, bufb)
    gsems = (gsem0, gsem1)
    iota = jax.lax.iota(jnp.int32, 16)
    bio = [iota + 16 * g for g in range(CB // 16)]
    diag = [lax.bitwise_and(iota + k, 15) for k in range(16)]

    def fire_gathers(s, h, buf, gsem):
        def grp_body(g, carry):
            v = idx_v[s, pl.ds(h * CB + g * 16, 16)]
            for j in range(16):
                i = v[j]
                pltpu.async_copy(table_hbm.at[pl.ds(i, 1)],
                                 buf.at[pl.ds(g * 16 + j, 1)], gsem)
            return carry
        lax.fori_loop(0, CB // 16, grp_body, 0)

    def drain_gathers(buf, gsem):
        pltpu.make_async_copy(table_hbm.at[pl.ds(0, CB)], buf, gsem).wait()

    def transpose(buf, h):
        lanes = [bio[g] + h * CB for g in range(CB // 16)]

        def d0_body(dblk, carry):
            d0 = dblk * 16
            for g in range(CB // 16):
                for k in range(16):
                    dcol = d0 + diag[k]
                    col = plsc.load_gather(buf, [bio[g], dcol])
                    plsc.store_scatter(tbuf, [dcol, lanes[g]], col)
            return carry
        lax.fori_loop(0, (DT * 8) // 16, d0_body, 0)

    def fire_write(s):
        base = s * (DT * BT * 8) + w * 8
        for dt in range(DT):
            pltpu.async_copy(tbuf.at[pl.ds(dt * 8, 8)],
                             out_hbm.at[pl.ds(base + dt * BT * 8, 8)], wsem)

    def drain_write():
        pltpu.make_async_copy(tbuf, out_hbm.at[pl.ds(0, DT * 8)], wsem).wait()

    fire_gathers(0, 0, bufa, gsem0)
    fire_gathers(0, 1, bufb, gsem1)

    def main_body(s, carry):
        drain_gathers(bufa, gsem0)

        @pl.when(s >= 1)
        def _():
            drain_write()
        transpose(bufa, 0)

        @pl.when(s <= S - 2)
        def _():
            fire_gathers(s + 1, 0, bufa, gsem0)
        drain_gathers(bufb, gsem1)
        transpose(bufb, 1)
        fire_write(s)

        @pl.when(s <= S - 2)
        def _():
            fire_gathers(s + 1, 1, bufb, gsem1)
        return carry

    lax.fori_loop(0, S, main_body, 0)
    drain_write()


def kernel(input, table):
    idx_t = input.astype(jnp.int32).T
    z2 = _emb_gather(idx_t, table)
    z5 = z2.reshape(S, DT, BT, 8, 128)
    t = jnp.transpose(z5, (2, 4, 0, 1, 3))
    return t.reshape(B, S, DT * 8)[:, :, :D]
